# Initial kernel scaffold; baseline (speedup 1.0000x reference)
import functools

import jax
import jax.numpy as jnp
from jax import lax
from jax.experimental import pallas as pl
from jax.experimental.pallas import tpu as pltpu
from jax.experimental.pallas import tpu_sc as plsc

N = 10000
D = 128
E = 320000
STEPS = 2

PAD = 16
DP = D + PAD

NS = 16
EPT = E // NS
CH = 80
NCHUNK = EPT // CH
RPT = N // NS

BR = 1250
GRID = N // BR


def _segsum_body(feat0, feat1, src0, dst0, src1, dst1, zinit,
                 out0, out1, idx_s, idx_d, rows, acc, sem):
    cid = lax.axis_index("c")
    sid = lax.axis_index("s")
    r0 = sid * RPT
    pltpu.sync_copy(zinit.at[pl.ds(r0, RPT)], acc.at[pl.ds(r0, RPT)])
    plsc.subcore_barrier()

    def run(feat, src, dst, out):
        def body(g, carry):
            off = sid * EPT + g * CH
            pltpu.sync_copy(src.at[pl.ds(off, CH)], idx_s)
            pltpu.sync_copy(dst.at[pl.ds(off, CH)], idx_d)
            pltpu.async_copy(feat.at[idx_s], rows, sem).wait()
            pltpu.sync_copy(rows, acc.at[idx_d], add=True)
            return carry
        lax.fori_loop(0, NCHUNK, body, 0)
        plsc.subcore_barrier()
        pltpu.sync_copy(acc.at[pl.ds(r0, RPT)], out.at[pl.ds(r0, RPT)])

    @pl.when(cid == 0)
    def _():
        run(feat0, src0, dst0, out0)

    @pl.when(cid == 1)
    def _():
        run(feat1, src1, dst1, out1)


_segsum = functools.partial(
    pl.kernel,
    mesh=plsc.VectorSubcoreMesh(core_axis_name="c", subcore_axis_name="s"),
    out_type=[jax.ShapeDtypeStruct((N, DP), jnp.float32)] * 2,
    scratch_types=[
        pltpu.VMEM((CH,), jnp.int32),
        pltpu.VMEM((CH,), jnp.int32),
        pltpu.VMEM((CH, DP), jnp.float32),
        pltpu.VMEM_SHARED((N, DP), jnp.float32),
        pltpu.SemaphoreType.DMA,
    ],
)(_segsum_body)


def _feat_body(h_ref, w0_ref, b0_ref, w1_ref, b1_ref, o0_ref, o1_ref):
    h = h_ref[...]
    one = jnp.ones((BR, PAD), jnp.float32)
    f0 = jnp.dot(h, w0_ref[...], preferred_element_type=jnp.float32) + b0_ref[...]
    o0_ref[...] = jnp.concatenate([f0, one], axis=1)
    f1 = jnp.dot(h, w1_ref[...], preferred_element_type=jnp.float32) + b1_ref[...]
    o1_ref[...] = jnp.concatenate([f1, one], axis=1)


_feat = pl.pallas_call(
    _feat_body,
    grid=(GRID,),
    in_specs=[
        pl.BlockSpec((BR, D), lambda i: (i, 0)),
        pl.BlockSpec((D, D), lambda i: (0, 0)),
        pl.BlockSpec((1, D), lambda i: (0, 0)),
        pl.BlockSpec((D, D), lambda i: (0, 0)),
        pl.BlockSpec((1, D), lambda i: (0, 0)),
    ],
    out_specs=[
        pl.BlockSpec((BR, DP), lambda i: (i, 0)),
        pl.BlockSpec((BR, DP), lambda i: (i, 0)),
    ],
    out_shape=[jax.ShapeDtypeStruct((N, DP), jnp.float32)] * 2,
)


def _gru_body(a0_ref, a1_ref, h_ref, wih_ref, whh_ref, bih_ref, bhh_ref, o_ref):
    c0 = jnp.max(a0_ref[:, D:], axis=1, keepdims=True)
    m0 = a0_ref[:, :D] / jnp.maximum(c0, 1.0)
    c1 = jnp.max(a1_ref[:, D:], axis=1, keepdims=True)
    m1 = a1_ref[:, :D] / jnp.maximum(c1, 1.0)
    agg = m0 + m1
    h = h_ref[...]
    gi = jnp.dot(agg, wih_ref[...], preferred_element_type=jnp.float32) + bih_ref[...]
    gh = jnp.dot(h, whh_ref[...], preferred_element_type=jnp.float32) + bhh_ref[...]
    r = jax.nn.sigmoid(gi[:, :D] + gh[:, :D])
    z = jax.nn.sigmoid(gi[:, D:2 * D] + gh[:, D:2 * D])
    n = jnp.tanh(gi[:, 2 * D:] + r * gh[:, 2 * D:])
    o_ref[...] = (1.0 - z) * n + z * h


_gru = pl.pallas_call(
    _gru_body,
    grid=(GRID,),
    in_specs=[
        pl.BlockSpec((BR, DP), lambda i: (i, 0)),
        pl.BlockSpec((BR, DP), lambda i: (i, 0)),
        pl.BlockSpec((BR, D), lambda i: (i, 0)),
        pl.BlockSpec((D, 3 * D), lambda i: (0, 0)),
        pl.BlockSpec((D, 3 * D), lambda i: (0, 0)),
        pl.BlockSpec((1, 3 * D), lambda i: (0, 0)),
        pl.BlockSpec((1, 3 * D), lambda i: (0, 0)),
    ],
    out_specs=pl.BlockSpec((BR, D), lambda i: (i, 0)),
    out_shape=jax.ShapeDtypeStruct((N, D), jnp.float32),
)


def kernel(x, edge_index_e0, edge_index_e1, W_e0, b_e0, W_e1, b_e1,
           W_ih, W_hh, b_ih, b_hh):
    src0, dst0 = edge_index_e0[0], edge_index_e0[1]
    src1, dst1 = edge_index_e1[0], edge_index_e1[1]
    w0t = W_e0.T
    w1t = W_e1.T
    wiht = W_ih.T
    whht = W_hh.T
    b0 = b_e0.reshape(1, D)
    b1 = b_e1.reshape(1, D)
    bih = b_ih.reshape(1, 3 * D)
    bhh = b_hh.reshape(1, 3 * D)
    zinit = jnp.zeros((N, DP), jnp.float32)

    h = x
    for _ in range(STEPS):
        f0, f1 = _feat(h, w0t, b0, w1t, b1)
        a0, a1 = _segsum(f0, f1, src0, dst0, src1, dst1, zinit)
        h = _gru(a0, a1, h, wiht, whht, bih, bhh)
    return h


# speedup vs baseline: 4.1677x; 4.1677x over previous
import functools

import jax
import jax.numpy as jnp
from jax import lax
from jax.experimental import pallas as pl
from jax.experimental.pallas import tpu as pltpu
from jax.experimental.pallas import tpu_sc as plsc

N = 10000
D = 128
E = 320000
STEPS = 2

PAD = 16
DP = D + PAD

NS = 16
EPT = E // NS
CH = 80
NCHUNK = EPT // CH
R_BIG = 632
R_LAST = N - 15 * R_BIG

BR = 2000
GRID = N // BR


def _segsum_body(feat0, feat1, src0, dst0, src1, dst1, zinit,
                 out0, out1, idx_s, idx_d, rows, acc, sem):
    cid = lax.axis_index("c")
    sid = lax.axis_index("s")
    r0 = pl.multiple_of(sid * R_BIG, 8)

    def _rowcopy(src_ref, dst_ref):
        @pl.when(sid < NS - 1)
        def _():
            pltpu.sync_copy(src_ref.at[pl.ds(r0, R_BIG)],
                            dst_ref.at[pl.ds(r0, R_BIG)])

        @pl.when(sid == NS - 1)
        def _():
            pltpu.sync_copy(src_ref.at[pl.ds(15 * R_BIG, R_LAST)],
                            dst_ref.at[pl.ds(15 * R_BIG, R_LAST)])

    _rowcopy(zinit, acc)
    plsc.subcore_barrier()

    def run(feat, src, dst, out):
        def body(g, carry):
            off = sid * EPT + g * CH
            pltpu.sync_copy(src.at[pl.ds(off, CH)], idx_s)
            pltpu.sync_copy(dst.at[pl.ds(off, CH)], idx_d)
            pltpu.async_copy(feat.at[idx_s], rows, sem).wait()
            pltpu.sync_copy(rows, acc.at[idx_d], add=True)
            return carry
        lax.fori_loop(0, NCHUNK, body, 0)
        plsc.subcore_barrier()
        _rowcopy(acc, out)

    @pl.when(cid == 0)
    def _():
        run(feat0, src0, dst0, out0)

    @pl.when(cid == 1)
    def _():
        run(feat1, src1, dst1, out1)


_segsum = functools.partial(
    pl.kernel,
    mesh=plsc.VectorSubcoreMesh(core_axis_name="c", subcore_axis_name="s"),
    compiler_params=pltpu.CompilerParams(use_tc_tiling_on_sc=False),
    out_type=[jax.ShapeDtypeStruct((N, DP), jnp.float32)] * 2,
    scratch_types=[
        pltpu.VMEM((CH,), jnp.int32),
        pltpu.VMEM((CH,), jnp.int32),
        pltpu.VMEM((CH, DP), jnp.float32),
        pltpu.VMEM_SHARED((N, DP), jnp.float32),
        pltpu.SemaphoreType.DMA,
    ],
)(_segsum_body)


def _feat_body(h_ref, w0_ref, b0_ref, w1_ref, b1_ref, o0_ref, o1_ref):
    h = h_ref[...]
    one = jnp.ones((BR, PAD), jnp.float32)
    f0 = jnp.dot(h, w0_ref[...], preferred_element_type=jnp.float32) + b0_ref[...]
    o0_ref[...] = jnp.concatenate([f0, one], axis=1)
    f1 = jnp.dot(h, w1_ref[...], preferred_element_type=jnp.float32) + b1_ref[...]
    o1_ref[...] = jnp.concatenate([f1, one], axis=1)


_feat = pl.pallas_call(
    _feat_body,
    grid=(GRID,),
    in_specs=[
        pl.BlockSpec((BR, D), lambda i: (i, 0)),
        pl.BlockSpec((D, D), lambda i: (0, 0)),
        pl.BlockSpec((1, D), lambda i: (0, 0)),
        pl.BlockSpec((D, D), lambda i: (0, 0)),
        pl.BlockSpec((1, D), lambda i: (0, 0)),
    ],
    out_specs=[
        pl.BlockSpec((BR, DP), lambda i: (i, 0)),
        pl.BlockSpec((BR, DP), lambda i: (i, 0)),
    ],
    out_shape=[jax.ShapeDtypeStruct((N, DP), jnp.float32)] * 2,
)


def _gru_body(a0_ref, a1_ref, h_ref, wih_ref, whh_ref, bih_ref, bhh_ref, o_ref):
    c0 = jnp.max(a0_ref[:, D:], axis=1, keepdims=True)
    m0 = a0_ref[:, :D] / jnp.maximum(c0, 1.0)
    c1 = jnp.max(a1_ref[:, D:], axis=1, keepdims=True)
    m1 = a1_ref[:, :D] / jnp.maximum(c1, 1.0)
    agg = m0 + m1
    h = h_ref[...]
    gi = jnp.dot(agg, wih_ref[...], preferred_element_type=jnp.float32) + bih_ref[...]
    gh = jnp.dot(h, whh_ref[...], preferred_element_type=jnp.float32) + bhh_ref[...]
    r = jax.nn.sigmoid(gi[:, :D] + gh[:, :D])
    z = jax.nn.sigmoid(gi[:, D:2 * D] + gh[:, D:2 * D])
    n = jnp.tanh(gi[:, 2 * D:] + r * gh[:, 2 * D:])
    o_ref[...] = (1.0 - z) * n + z * h


_gru = pl.pallas_call(
    _gru_body,
    grid=(GRID,),
    in_specs=[
        pl.BlockSpec((BR, DP), lambda i: (i, 0)),
        pl.BlockSpec((BR, DP), lambda i: (i, 0)),
        pl.BlockSpec((BR, D), lambda i: (i, 0)),
        pl.BlockSpec((D, 3 * D), lambda i: (0, 0)),
        pl.BlockSpec((D, 3 * D), lambda i: (0, 0)),
        pl.BlockSpec((1, 3 * D), lambda i: (0, 0)),
        pl.BlockSpec((1, 3 * D), lambda i: (0, 0)),
    ],
    out_specs=pl.BlockSpec((BR, D), lambda i: (i, 0)),
    out_shape=jax.ShapeDtypeStruct((N, D), jnp.float32),
)


def kernel(x, edge_index_e0, edge_index_e1, W_e0, b_e0, W_e1, b_e1,
           W_ih, W_hh, b_ih, b_hh):
    src0, dst0 = edge_index_e0[0], edge_index_e0[1]
    src1, dst1 = edge_index_e1[0], edge_index_e1[1]
    w0t = W_e0.T
    w1t = W_e1.T
    wiht = W_ih.T
    whht = W_hh.T
    b0 = b_e0.reshape(1, D)
    b1 = b_e1.reshape(1, D)
    bih = b_ih.reshape(1, 3 * D)
    bhh = b_hh.reshape(1, 3 * D)
    zinit = jnp.zeros((N, DP), jnp.float32)

    h = x
    for _ in range(STEPS):
        f0, f1 = _feat(h, w0t, b0, w1t, b1)
        a0, a1 = _segsum(f0, f1, src0, dst0, src1, dst1, zinit)
        h = _gru(a0, a1, h, wiht, whht, bih, bhh)
    return h


# speedup vs baseline: 8.9343x; 2.1437x over previous
import functools

import jax
import jax.numpy as jnp
from jax import lax
from jax.experimental import pallas as pl
from jax.experimental.pallas import tpu as pltpu
from jax.experimental.pallas import tpu_sc as plsc

N = 10000
D = 128
E = 320000
STEPS = 2

PAD = 16
DP = D + PAD

NS = 16
EPT = E // NS
CH = 80
NCHUNK = EPT // CH
NB = 50
NBLK = NCHUNK // NB
NPAIRB = NB // 2
R_BIG = 632
R_LAST = N - 15 * R_BIG

BR = 2000
GRID = N // BR


def _segsum_body(feat0, feat1, src0, dst0, src1, dst1, zinit,
                 out0, out1, src_blk, dst_blk, rows_a, rows_b,
                 acc, sem_a, sem_b):
    cid = lax.axis_index("c")
    sid = lax.axis_index("s")
    r0 = pl.multiple_of(sid * R_BIG, 8)

    def _rowcopy(src_ref, dst_ref):
        @pl.when(sid < NS - 1)
        def _():
            pltpu.sync_copy(src_ref.at[pl.ds(r0, R_BIG)],
                            dst_ref.at[pl.ds(r0, R_BIG)])

        @pl.when(sid == NS - 1)
        def _():
            pltpu.sync_copy(src_ref.at[pl.ds(15 * R_BIG, R_LAST)],
                            dst_ref.at[pl.ds(15 * R_BIG, R_LAST)])

    _rowcopy(zinit, acc)
    plsc.subcore_barrier()

    def run(feat, src, dst, out):
        def block(j, carry):
            pltpu.sync_copy(src.at[sid, j], src_blk)
            pltpu.sync_copy(dst.at[sid, j], dst_blk)
            pltpu.async_copy(feat.at[src_blk.at[0]], rows_a, sem_a)

            def body(k, c):
                g0 = 2 * k
                pltpu.async_copy(feat.at[src_blk.at[g0 + 1]], rows_b, sem_b)
                pltpu.make_async_copy(feat.at[src_blk.at[g0]], rows_a, sem_a).wait()
                pltpu.sync_copy(rows_a, acc.at[dst_blk.at[g0]], add=True)

                @pl.when(k + 1 < NPAIRB)
                def _():
                    pltpu.async_copy(feat.at[src_blk.at[g0 + 2]], rows_a, sem_a)

                pltpu.make_async_copy(feat.at[src_blk.at[g0 + 1]], rows_b, sem_b).wait()
                pltpu.sync_copy(rows_b, acc.at[dst_blk.at[g0 + 1]], add=True)
                return c

            lax.fori_loop(0, NPAIRB, body, 0)
            return carry

        lax.fori_loop(0, NBLK, block, 0)
        plsc.subcore_barrier()
        _rowcopy(acc, out)

    @pl.when(cid == 0)
    def _():
        run(feat0, src0, dst0, out0)

    @pl.when(cid == 1)
    def _():
        run(feat1, src1, dst1, out1)


_segsum = functools.partial(
    pl.kernel,
    mesh=plsc.VectorSubcoreMesh(core_axis_name="c", subcore_axis_name="s"),
    compiler_params=pltpu.CompilerParams(use_tc_tiling_on_sc=False),
    out_type=[jax.ShapeDtypeStruct((N, DP), jnp.float32)] * 2,
    scratch_types=[
        pltpu.VMEM((NB, CH), jnp.int32),
        pltpu.VMEM((NB, CH), jnp.int32),
        pltpu.VMEM((CH, DP), jnp.float32),
        pltpu.VMEM((CH, DP), jnp.float32),
        pltpu.VMEM_SHARED((N, DP), jnp.float32),
        pltpu.SemaphoreType.DMA,
        pltpu.SemaphoreType.DMA,
    ],
)(_segsum_body)


def _feat_body(h_ref, w0_ref, b0_ref, w1_ref, b1_ref, o0_ref, o1_ref):
    h = h_ref[...]
    one = jnp.ones((BR, PAD), jnp.float32)
    f0 = jnp.dot(h, w0_ref[...], preferred_element_type=jnp.float32) + b0_ref[...]
    o0_ref[...] = jnp.concatenate([f0, one], axis=1)
    f1 = jnp.dot(h, w1_ref[...], preferred_element_type=jnp.float32) + b1_ref[...]
    o1_ref[...] = jnp.concatenate([f1, one], axis=1)


_feat = pl.pallas_call(
    _feat_body,
    grid=(GRID,),
    in_specs=[
        pl.BlockSpec((BR, D), lambda i: (i, 0)),
        pl.BlockSpec((D, D), lambda i: (0, 0)),
        pl.BlockSpec((1, D), lambda i: (0, 0)),
        pl.BlockSpec((D, D), lambda i: (0, 0)),
        pl.BlockSpec((1, D), lambda i: (0, 0)),
    ],
    out_specs=[
        pl.BlockSpec((BR, DP), lambda i: (i, 0)),
        pl.BlockSpec((BR, DP), lambda i: (i, 0)),
    ],
    out_shape=[jax.ShapeDtypeStruct((N, DP), jnp.float32)] * 2,
)


def _gru_body(a0_ref, a1_ref, h_ref, wih_ref, whh_ref, bih_ref, bhh_ref, o_ref):
    c0 = jnp.max(a0_ref[:, D:], axis=1, keepdims=True)
    m0 = a0_ref[:, :D] / jnp.maximum(c0, 1.0)
    c1 = jnp.max(a1_ref[:, D:], axis=1, keepdims=True)
    m1 = a1_ref[:, :D] / jnp.maximum(c1, 1.0)
    agg = m0 + m1
    h = h_ref[...]
    gi = jnp.dot(agg, wih_ref[...], preferred_element_type=jnp.float32) + bih_ref[...]
    gh = jnp.dot(h, whh_ref[...], preferred_element_type=jnp.float32) + bhh_ref[...]
    r = jax.nn.sigmoid(gi[:, :D] + gh[:, :D])
    z = jax.nn.sigmoid(gi[:, D:2 * D] + gh[:, D:2 * D])
    n = jnp.tanh(gi[:, 2 * D:] + r * gh[:, 2 * D:])
    o_ref[...] = (1.0 - z) * n + z * h


_gru = pl.pallas_call(
    _gru_body,
    grid=(GRID,),
    in_specs=[
        pl.BlockSpec((BR, DP), lambda i: (i, 0)),
        pl.BlockSpec((BR, DP), lambda i: (i, 0)),
        pl.BlockSpec((BR, D), lambda i: (i, 0)),
        pl.BlockSpec((D, 3 * D), lambda i: (0, 0)),
        pl.BlockSpec((D, 3 * D), lambda i: (0, 0)),
        pl.BlockSpec((1, 3 * D), lambda i: (0, 0)),
        pl.BlockSpec((1, 3 * D), lambda i: (0, 0)),
    ],
    out_specs=pl.BlockSpec((BR, D), lambda i: (i, 0)),
    out_shape=jax.ShapeDtypeStruct((N, D), jnp.float32),
)


def kernel(x, edge_index_e0, edge_index_e1, W_e0, b_e0, W_e1, b_e1,
           W_ih, W_hh, b_ih, b_hh):
    src0 = edge_index_e0[0].reshape(NS, NBLK, NB, CH)
    dst0 = edge_index_e0[1].reshape(NS, NBLK, NB, CH)
    src1 = edge_index_e1[0].reshape(NS, NBLK, NB, CH)
    dst1 = edge_index_e1[1].reshape(NS, NBLK, NB, CH)
    w0t = W_e0.T
    w1t = W_e1.T
    wiht = W_ih.T
    whht = W_hh.T
    b0 = b_e0.reshape(1, D)
    b1 = b_e1.reshape(1, D)
    bih = b_ih.reshape(1, 3 * D)
    bhh = b_hh.reshape(1, 3 * D)
    zinit = jnp.zeros((N, DP), jnp.float32)

    h = x
    for _ in range(STEPS):
        f0, f1 = _feat(h, w0t, b0, w1t, b1)
        a0, a1 = _segsum(f0, f1, src0, dst0, src1, dst1, zinit)
        h = _gru(a0, a1, h, wiht, whht, bih, bhh)
    return h


# speedup vs baseline: 10.4531x; 1.1700x over previous
import functools

import jax
import jax.numpy as jnp
from jax import lax
from jax.experimental import pallas as pl
from jax.experimental.pallas import tpu as pltpu
from jax.experimental.pallas import tpu_sc as plsc

N = 10000
D = 128
E = 320000
STEPS = 2

CW = 16

NS = 16
EPT = E // NS
CH = 80
NCHUNK = EPT // CH
NB = 50
NBLK = NCHUNK // NB
NPAIRB = NB // 2
R_BIG = 632
R_LAST = N - 15 * R_BIG

BR = 2000
GRID = N // BR


def _segsum_pipeline(sid, feat, src, dst, acc, src_blk, dst_blk,
                     rows_a, rows_b, sem_a, sem_b, cnt, ones_v):
    def block(j, carry):
        pltpu.sync_copy(src.at[sid, j], src_blk)
        pltpu.sync_copy(dst.at[sid, j], dst_blk)
        pltpu.async_copy(feat.at[src_blk.at[0]], rows_a, sem_a)

        def body(k, c):
            g0 = 2 * k
            pltpu.async_copy(feat.at[src_blk.at[g0 + 1]], rows_b, sem_b)
            pltpu.make_async_copy(feat.at[src_blk.at[g0]], rows_a, sem_a).wait()
            pltpu.sync_copy(rows_a, acc.at[dst_blk.at[g0]], add=True)
            if cnt is not None:
                pltpu.sync_copy(ones_v, cnt.at[dst_blk.at[g0]], add=True)

            @pl.when(k + 1 < NPAIRB)
            def _():
                pltpu.async_copy(feat.at[src_blk.at[g0 + 2]], rows_a, sem_a)

            pltpu.make_async_copy(feat.at[src_blk.at[g0 + 1]], rows_b, sem_b).wait()
            pltpu.sync_copy(rows_b, acc.at[dst_blk.at[g0 + 1]], add=True)
            if cnt is not None:
                pltpu.sync_copy(ones_v, cnt.at[dst_blk.at[g0 + 1]], add=True)
            return c

        lax.fori_loop(0, NPAIRB, body, 0)
        return carry

    lax.fori_loop(0, NBLK, block, 0)


def _rowcopy(sid, src_ref, dst_ref):
    r0 = pl.multiple_of(sid * R_BIG, 8)

    @pl.when(sid < NS - 1)
    def _():
        pltpu.sync_copy(src_ref.at[pl.ds(r0, R_BIG)],
                        dst_ref.at[pl.ds(r0, R_BIG)])

    @pl.when(sid == NS - 1)
    def _():
        pltpu.sync_copy(src_ref.at[pl.ds(15 * R_BIG, R_LAST)],
                        dst_ref.at[pl.ds(15 * R_BIG, R_LAST)])


def _segsum_cnt_body(feat0, feat1, src0, dst0, src1, dst1, zinit, zinit_c,
                     ones16, out0, out1, cnt0, cnt1,
                     src_blk, dst_blk, rows_a, rows_b, ones_v,
                     acc, cacc, sem_a, sem_b):
    cid = lax.axis_index("c")
    sid = lax.axis_index("s")
    _rowcopy(sid, zinit, acc)
    _rowcopy(sid, zinit_c, cacc)
    pltpu.sync_copy(ones16, ones_v)
    plsc.subcore_barrier()

    @pl.when(cid == 0)
    def _():
        _segsum_pipeline(sid, feat0, src0, dst0, acc, src_blk, dst_blk,
                         rows_a, rows_b, sem_a, sem_b, cacc, ones_v)
        plsc.subcore_barrier()
        _rowcopy(sid, acc, out0)
        _rowcopy(sid, cacc, cnt0)

    @pl.when(cid == 1)
    def _():
        _segsum_pipeline(sid, feat1, src1, dst1, acc, src_blk, dst_blk,
                         rows_a, rows_b, sem_a, sem_b, cacc, ones_v)
        plsc.subcore_barrier()
        _rowcopy(sid, acc, out1)
        _rowcopy(sid, cacc, cnt1)


def _segsum_nc_body(feat0, feat1, src0, dst0, src1, dst1, zinit,
                    out0, out1,
                    src_blk, dst_blk, rows_a, rows_b,
                    acc, sem_a, sem_b):
    cid = lax.axis_index("c")
    sid = lax.axis_index("s")
    _rowcopy(sid, zinit, acc)
    plsc.subcore_barrier()

    @pl.when(cid == 0)
    def _():
        _segsum_pipeline(sid, feat0, src0, dst0, acc, src_blk, dst_blk,
                         rows_a, rows_b, sem_a, sem_b, None, None)
        plsc.subcore_barrier()
        _rowcopy(sid, acc, out0)

    @pl.when(cid == 1)
    def _():
        _segsum_pipeline(sid, feat1, src1, dst1, acc, src_blk, dst_blk,
                         rows_a, rows_b, sem_a, sem_b, None, None)
        plsc.subcore_barrier()
        _rowcopy(sid, acc, out1)


_SEG_MESH = plsc.VectorSubcoreMesh(core_axis_name="c", subcore_axis_name="s")

_segsum_cnt = functools.partial(
    pl.kernel,
    mesh=_SEG_MESH,
    compiler_params=pltpu.CompilerParams(use_tc_tiling_on_sc=False),
    out_type=[jax.ShapeDtypeStruct((N, D), jnp.float32)] * 2
    + [jax.ShapeDtypeStruct((N, CW), jnp.float32)] * 2,
    scratch_types=[
        pltpu.VMEM((NB, CH), jnp.int32),
        pltpu.VMEM((NB, CH), jnp.int32),
        pltpu.VMEM((CH, D), jnp.float32),
        pltpu.VMEM((CH, D), jnp.float32),
        pltpu.VMEM((CH, CW), jnp.float32),
        pltpu.VMEM_SHARED((N, D), jnp.float32),
        pltpu.VMEM_SHARED((N, CW), jnp.float32),
        pltpu.SemaphoreType.DMA,
        pltpu.SemaphoreType.DMA,
    ],
)(_segsum_cnt_body)

_segsum_nc = functools.partial(
    pl.kernel,
    mesh=_SEG_MESH,
    compiler_params=pltpu.CompilerParams(use_tc_tiling_on_sc=False),
    out_type=[jax.ShapeDtypeStruct((N, D), jnp.float32)] * 2,
    scratch_types=[
        pltpu.VMEM((NB, CH), jnp.int32),
        pltpu.VMEM((NB, CH), jnp.int32),
        pltpu.VMEM((CH, D), jnp.float32),
        pltpu.VMEM((CH, D), jnp.float32),
        pltpu.VMEM_SHARED((N, D), jnp.float32),
        pltpu.SemaphoreType.DMA,
        pltpu.SemaphoreType.DMA,
    ],
)(_segsum_nc_body)


def _feat_body(h_ref, w0_ref, b0_ref, w1_ref, b1_ref, o0_ref, o1_ref):
    h = h_ref[...]
    o0_ref[...] = jnp.dot(h, w0_ref[...],
                          preferred_element_type=jnp.float32) + b0_ref[...]
    o1_ref[...] = jnp.dot(h, w1_ref[...],
                          preferred_element_type=jnp.float32) + b1_ref[...]


_feat = pl.pallas_call(
    _feat_body,
    grid=(GRID,),
    in_specs=[
        pl.BlockSpec((BR, D), lambda i: (i, 0)),
        pl.BlockSpec((D, D), lambda i: (0, 0)),
        pl.BlockSpec((1, D), lambda i: (0, 0)),
        pl.BlockSpec((D, D), lambda i: (0, 0)),
        pl.BlockSpec((1, D), lambda i: (0, 0)),
    ],
    out_specs=[
        pl.BlockSpec((BR, D), lambda i: (i, 0)),
        pl.BlockSpec((BR, D), lambda i: (i, 0)),
    ],
    out_shape=[jax.ShapeDtypeStruct((N, D), jnp.float32)] * 2,
)


def _gru_body(a0_ref, a1_ref, c0_ref, c1_ref, h_ref,
              wih_ref, whh_ref, bih_ref, bhh_ref, o_ref):
    c0 = jnp.max(c0_ref[...], axis=1, keepdims=True)
    c1 = jnp.max(c1_ref[...], axis=1, keepdims=True)
    m0 = a0_ref[...] / jnp.maximum(c0, 1.0)
    m1 = a1_ref[...] / jnp.maximum(c1, 1.0)
    agg = m0 + m1
    h = h_ref[...]
    gi = jnp.dot(agg, wih_ref[...],
                 preferred_element_type=jnp.float32) + bih_ref[...]
    gh = jnp.dot(h, whh_ref[...],
                 preferred_element_type=jnp.float32) + bhh_ref[...]
    r = jax.nn.sigmoid(gi[:, :D] + gh[:, :D])
    z = jax.nn.sigmoid(gi[:, D:2 * D] + gh[:, D:2 * D])
    n = jnp.tanh(gi[:, 2 * D:] + r * gh[:, 2 * D:])
    o_ref[...] = (1.0 - z) * n + z * h


_gru = pl.pallas_call(
    _gru_body,
    grid=(GRID,),
    in_specs=[
        pl.BlockSpec((BR, D), lambda i: (i, 0)),
        pl.BlockSpec((BR, D), lambda i: (i, 0)),
        pl.BlockSpec((BR, CW), lambda i: (i, 0)),
        pl.BlockSpec((BR, CW), lambda i: (i, 0)),
        pl.BlockSpec((BR, D), lambda i: (i, 0)),
        pl.BlockSpec((D, 3 * D), lambda i: (0, 0)),
        pl.BlockSpec((D, 3 * D), lambda i: (0, 0)),
        pl.BlockSpec((1, 3 * D), lambda i: (0, 0)),
        pl.BlockSpec((1, 3 * D), lambda i: (0, 0)),
    ],
    out_specs=pl.BlockSpec((BR, D), lambda i: (i, 0)),
    out_shape=jax.ShapeDtypeStruct((N, D), jnp.float32),
)


def kernel(x, edge_index_e0, edge_index_e1, W_e0, b_e0, W_e1, b_e1,
           W_ih, W_hh, b_ih, b_hh):
    src0 = edge_index_e0[0].reshape(NS, NBLK, NB, CH)
    dst0 = edge_index_e0[1].reshape(NS, NBLK, NB, CH)
    src1 = edge_index_e1[0].reshape(NS, NBLK, NB, CH)
    dst1 = edge_index_e1[1].reshape(NS, NBLK, NB, CH)
    w0t = W_e0.T
    w1t = W_e1.T
    wiht = W_ih.T
    whht = W_hh.T
    b0 = b_e0.reshape(1, D)
    b1 = b_e1.reshape(1, D)
    bih = b_ih.reshape(1, 3 * D)
    bhh = b_hh.reshape(1, 3 * D)
    zinit = jnp.zeros((N, D), jnp.float32)
    zinit_c = jnp.zeros((N, CW), jnp.float32)
    ones16 = jnp.ones((CH, CW), jnp.float32)

    h = x
    cnt0 = cnt1 = None
    for step in range(STEPS):
        f0, f1 = _feat(h, w0t, b0, w1t, b1)
        if step == 0:
            a0, a1, cnt0, cnt1 = _segsum_cnt(
                f0, f1, src0, dst0, src1, dst1, zinit, zinit_c, ones16)
        else:
            a0, a1 = _segsum_nc(f0, f1, src0, dst0, src1, dst1, zinit)
        h = _gru(a0, a1, cnt0, cnt1, h, wiht, whht, bih, bhh)
    return h


# speedup vs baseline: 10.9556x; 1.0481x over previous
import functools

import jax
import jax.numpy as jnp
from jax import lax
from jax.experimental import pallas as pl
from jax.experimental.pallas import tpu as pltpu
from jax.experimental.pallas import tpu_sc as plsc

N = 10000
D = 128
E = 320000
STEPS = 2

CW = 16

NS = 16
EPT = E // NS
CH = 80
NCHUNK = EPT // CH
NB = 50
NBLK = NCHUNK // NB
NPAIRB = NB // 2
R_BIG = 632
R_LAST = N - 15 * R_BIG

BR = 2000
GRID = N // BR


def _segsum_pipeline(sid, feat, ei, acc, src_blk, dst_blk,
                     rows_a, rows_b, sem_a, sem_b, cnt, ones_v):
    def block(j, carry):
        pltpu.sync_copy(ei.at[0, sid, j], src_blk)
        pltpu.sync_copy(ei.at[1, sid, j], dst_blk)
        pltpu.async_copy(feat.at[src_blk.at[0]], rows_a, sem_a)

        def body(k, c):
            g0 = 2 * k
            pltpu.async_copy(feat.at[src_blk.at[g0 + 1]], rows_b, sem_b)
            pltpu.make_async_copy(feat.at[src_blk.at[g0]], rows_a, sem_a).wait()
            pltpu.sync_copy(rows_a, acc.at[dst_blk.at[g0]], add=True)
            if cnt is not None:
                pltpu.sync_copy(ones_v, cnt.at[dst_blk.at[g0]], add=True)

            @pl.when(k + 1 < NPAIRB)
            def _():
                pltpu.async_copy(feat.at[src_blk.at[g0 + 2]], rows_a, sem_a)

            pltpu.make_async_copy(feat.at[src_blk.at[g0 + 1]], rows_b, sem_b).wait()
            pltpu.sync_copy(rows_b, acc.at[dst_blk.at[g0 + 1]], add=True)
            if cnt is not None:
                pltpu.sync_copy(ones_v, cnt.at[dst_blk.at[g0 + 1]], add=True)
            return c

        lax.fori_loop(0, NPAIRB, body, 0)
        return carry

    lax.fori_loop(0, NBLK, block, 0)


def _rowcopy(sid, src_ref, dst_ref):
    r0 = pl.multiple_of(sid * R_BIG, 8)

    @pl.when(sid < NS - 1)
    def _():
        pltpu.sync_copy(src_ref.at[pl.ds(r0, R_BIG)],
                        dst_ref.at[pl.ds(r0, R_BIG)])

    @pl.when(sid == NS - 1)
    def _():
        pltpu.sync_copy(src_ref.at[pl.ds(15 * R_BIG, R_LAST)],
                        dst_ref.at[pl.ds(15 * R_BIG, R_LAST)])


def _segsum_cnt_body(feat0, feat1, ei0, ei1, zinit, zinit_c,
                     ones16, out0, out1, cnt0, cnt1,
                     src_blk, dst_blk, rows_a, rows_b, ones_v,
                     acc, cacc, sem_a, sem_b):
    cid = lax.axis_index("c")
    sid = lax.axis_index("s")
    _rowcopy(sid, zinit, acc)
    _rowcopy(sid, zinit_c, cacc)
    pltpu.sync_copy(ones16, ones_v)
    plsc.subcore_barrier()

    @pl.when(cid == 0)
    def _():
        _segsum_pipeline(sid, feat0, ei0, acc, src_blk, dst_blk,
                         rows_a, rows_b, sem_a, sem_b, cacc, ones_v)
        plsc.subcore_barrier()
        _rowcopy(sid, acc, out0)
        _rowcopy(sid, cacc, cnt0)

    @pl.when(cid == 1)
    def _():
        _segsum_pipeline(sid, feat1, ei1, acc, src_blk, dst_blk,
                         rows_a, rows_b, sem_a, sem_b, cacc, ones_v)
        plsc.subcore_barrier()
        _rowcopy(sid, acc, out1)
        _rowcopy(sid, cacc, cnt1)


def _segsum_nc_body(feat0, feat1, ei0, ei1, zinit,
                    out0, out1,
                    src_blk, dst_blk, rows_a, rows_b,
                    acc, sem_a, sem_b):
    cid = lax.axis_index("c")
    sid = lax.axis_index("s")
    _rowcopy(sid, zinit, acc)
    plsc.subcore_barrier()

    @pl.when(cid == 0)
    def _():
        _segsum_pipeline(sid, feat0, ei0, acc, src_blk, dst_blk,
                         rows_a, rows_b, sem_a, sem_b, None, None)
        plsc.subcore_barrier()
        _rowcopy(sid, acc, out0)

    @pl.when(cid == 1)
    def _():
        _segsum_pipeline(sid, feat1, ei1, acc, src_blk, dst_blk,
                         rows_a, rows_b, sem_a, sem_b, None, None)
        plsc.subcore_barrier()
        _rowcopy(sid, acc, out1)


_SEG_MESH = plsc.VectorSubcoreMesh(core_axis_name="c", subcore_axis_name="s")

_segsum_cnt = functools.partial(
    pl.kernel,
    mesh=_SEG_MESH,
    compiler_params=pltpu.CompilerParams(use_tc_tiling_on_sc=False),
    out_type=[jax.ShapeDtypeStruct((N, D), jnp.float32)] * 2
    + [jax.ShapeDtypeStruct((N, CW), jnp.float32)] * 2,
    scratch_types=[
        pltpu.VMEM((NB, CH), jnp.int32),
        pltpu.VMEM((NB, CH), jnp.int32),
        pltpu.VMEM((CH, D), jnp.float32),
        pltpu.VMEM((CH, D), jnp.float32),
        pltpu.VMEM((CH, CW), jnp.float32),
        pltpu.VMEM_SHARED((N, D), jnp.float32),
        pltpu.VMEM_SHARED((N, CW), jnp.float32),
        pltpu.SemaphoreType.DMA,
        pltpu.SemaphoreType.DMA,
    ],
)(_segsum_cnt_body)

_segsum_nc = functools.partial(
    pl.kernel,
    mesh=_SEG_MESH,
    compiler_params=pltpu.CompilerParams(use_tc_tiling_on_sc=False),
    out_type=[jax.ShapeDtypeStruct((N, D), jnp.float32)] * 2,
    scratch_types=[
        pltpu.VMEM((NB, CH), jnp.int32),
        pltpu.VMEM((NB, CH), jnp.int32),
        pltpu.VMEM((CH, D), jnp.float32),
        pltpu.VMEM((CH, D), jnp.float32),
        pltpu.VMEM_SHARED((N, D), jnp.float32),
        pltpu.SemaphoreType.DMA,
        pltpu.SemaphoreType.DMA,
    ],
)(_segsum_nc_body)


def _feat_body(h_ref, w0_ref, b0_ref, w1_ref, b1_ref, o0_ref, o1_ref):
    h = h_ref[...]
    o0_ref[...] = jnp.dot(h, w0_ref[...],
                          preferred_element_type=jnp.float32) + b0_ref[...]
    o1_ref[...] = jnp.dot(h, w1_ref[...],
                          preferred_element_type=jnp.float32) + b1_ref[...]


_feat = pl.pallas_call(
    _feat_body,
    grid=(GRID,),
    in_specs=[
        pl.BlockSpec((BR, D), lambda i: (i, 0)),
        pl.BlockSpec((D, D), lambda i: (0, 0)),
        pl.BlockSpec((1, D), lambda i: (0, 0)),
        pl.BlockSpec((D, D), lambda i: (0, 0)),
        pl.BlockSpec((1, D), lambda i: (0, 0)),
    ],
    out_specs=[
        pl.BlockSpec((BR, D), lambda i: (i, 0)),
        pl.BlockSpec((BR, D), lambda i: (i, 0)),
    ],
    out_shape=[jax.ShapeDtypeStruct((N, D), jnp.float32)] * 2,
)


def _gru_core(a0_ref, a1_ref, c0_ref, c1_ref, h_ref,
              wih_ref, whh_ref, bih_ref, bhh_ref):
    c0 = jnp.max(c0_ref[...], axis=1, keepdims=True)
    c1 = jnp.max(c1_ref[...], axis=1, keepdims=True)
    m0 = a0_ref[...] / jnp.maximum(c0, 1.0)
    m1 = a1_ref[...] / jnp.maximum(c1, 1.0)
    agg = m0 + m1
    h = h_ref[...]
    gi = jnp.dot(agg, wih_ref[...],
                 preferred_element_type=jnp.float32) + bih_ref[...]
    gh = jnp.dot(h, whh_ref[...],
                 preferred_element_type=jnp.float32) + bhh_ref[...]
    r = jax.nn.sigmoid(gi[:, :D] + gh[:, :D])
    z = jax.nn.sigmoid(gi[:, D:2 * D] + gh[:, D:2 * D])
    n = jnp.tanh(gi[:, 2 * D:] + r * gh[:, 2 * D:])
    return (1.0 - z) * n + z * h


def _gru_body(a0_ref, a1_ref, c0_ref, c1_ref, h_ref,
              wih_ref, whh_ref, bih_ref, bhh_ref, o_ref):
    o_ref[...] = _gru_core(a0_ref, a1_ref, c0_ref, c1_ref, h_ref,
                           wih_ref, whh_ref, bih_ref, bhh_ref)


def _grufeat_body(a0_ref, a1_ref, c0_ref, c1_ref, h_ref,
                  wih_ref, whh_ref, bih_ref, bhh_ref,
                  w0_ref, b0_ref, w1_ref, b1_ref,
                  o_ref, o0_ref, o1_ref):
    hn = _gru_core(a0_ref, a1_ref, c0_ref, c1_ref, h_ref,
                   wih_ref, whh_ref, bih_ref, bhh_ref)
    o_ref[...] = hn
    o0_ref[...] = jnp.dot(hn, w0_ref[...],
                          preferred_element_type=jnp.float32) + b0_ref[...]
    o1_ref[...] = jnp.dot(hn, w1_ref[...],
                          preferred_element_type=jnp.float32) + b1_ref[...]


_GRU_IN_SPECS = [
    pl.BlockSpec((BR, D), lambda i: (i, 0)),
    pl.BlockSpec((BR, D), lambda i: (i, 0)),
    pl.BlockSpec((BR, CW), lambda i: (i, 0)),
    pl.BlockSpec((BR, CW), lambda i: (i, 0)),
    pl.BlockSpec((BR, D), lambda i: (i, 0)),
    pl.BlockSpec((D, 3 * D), lambda i: (0, 0)),
    pl.BlockSpec((D, 3 * D), lambda i: (0, 0)),
    pl.BlockSpec((1, 3 * D), lambda i: (0, 0)),
    pl.BlockSpec((1, 3 * D), lambda i: (0, 0)),
]

_gru = pl.pallas_call(
    _gru_body,
    grid=(GRID,),
    in_specs=_GRU_IN_SPECS,
    out_specs=pl.BlockSpec((BR, D), lambda i: (i, 0)),
    out_shape=jax.ShapeDtypeStruct((N, D), jnp.float32),
)

_grufeat = pl.pallas_call(
    _grufeat_body,
    grid=(GRID,),
    in_specs=_GRU_IN_SPECS + [
        pl.BlockSpec((D, D), lambda i: (0, 0)),
        pl.BlockSpec((1, D), lambda i: (0, 0)),
        pl.BlockSpec((D, D), lambda i: (0, 0)),
        pl.BlockSpec((1, D), lambda i: (0, 0)),
    ],
    out_specs=[
        pl.BlockSpec((BR, D), lambda i: (i, 0)),
        pl.BlockSpec((BR, D), lambda i: (i, 0)),
        pl.BlockSpec((BR, D), lambda i: (i, 0)),
    ],
    out_shape=[jax.ShapeDtypeStruct((N, D), jnp.float32)] * 3,
)


def kernel(x, edge_index_e0, edge_index_e1, W_e0, b_e0, W_e1, b_e1,
           W_ih, W_hh, b_ih, b_hh):
    ei0 = edge_index_e0.reshape(2, NS, NBLK, NB, CH)
    ei1 = edge_index_e1.reshape(2, NS, NBLK, NB, CH)
    w0t = W_e0.T
    w1t = W_e1.T
    wiht = W_ih.T
    whht = W_hh.T
    b0 = b_e0.reshape(1, D)
    b1 = b_e1.reshape(1, D)
    bih = b_ih.reshape(1, 3 * D)
    bhh = b_hh.reshape(1, 3 * D)
    zinit = jnp.zeros((N, D), jnp.float32)
    zinit_c = jnp.zeros((N, CW), jnp.float32)
    ones16 = jnp.ones((CH, CW), jnp.float32)

    f0, f1 = _feat(x, w0t, b0, w1t, b1)
    a0, a1, cnt0, cnt1 = _segsum_cnt(f0, f1, ei0, ei1, zinit, zinit_c, ones16)
    h, f0, f1 = _grufeat(a0, a1, cnt0, cnt1, x, wiht, whht, bih, bhh,
                         w0t, b0, w1t, b1)
    a0, a1 = _segsum_nc(f0, f1, ei0, ei1, zinit)
    return _gru(a0, a1, cnt0, cnt1, h, wiht, whht, bih, bhh)
